# baseline (speedup 1.0000x reference)
import jax
import jax.numpy as jnp
from jax import lax
from jax.experimental import pallas as pl
from jax.experimental.pallas import tpu as pltpu
from jax.experimental.pallas import tpu_sc as plsc

B = 8
V = 100000
W = 3200
WLAST = 768
EDGE0 = 31 * W + WLAST
EDGE = V - EDGE0
NW = 32
UNROLL = 8
IMAX = 2**31 - 1
CH = 1536


def _chunk_argmax(log_v, pen_v, r, col0, lo, hi, lane, carry):
    def step(j, c):
        bv, bi = c
        off = pl.multiple_of(j * 16, 16)
        sc = log_v[r, pl.ds(off, 16)] * pen_v[r, pl.ds(off, 16)]
        gi = col0 + off + lane
        take = sc > bv
        return jnp.where(take, sc, bv), jnp.where(take, gi, bi)

    return lax.fori_loop(lo // 16, hi // 16, step, carry, unroll=UNROLL)


def _body(log_h, pen_h, pv_h, mode_h, candv_h, candi_h, out_h, idxw_h,
          log_v, pen_v, lt_v, pt_v, sv_v, si_v,
          mv_v, mi_v, stv_v, sti_v, pv_v, pw_v, pe_v, ist_v,
          sem_l, sem_p, sem_l2, sem_p2, sem_a, sem_b):
    c = lax.axis_index("c")
    s = lax.axis_index("s")
    w = c * 16 + s
    lane = jnp.arange(16, dtype=jnp.int32)
    neg_inf = jnp.full((16,), -jnp.inf, dtype=jnp.float32)
    pltpu.sync_copy(mode_h, ist_v.at[pl.ds(0, 1)])
    mode = ist_v[pl.ds(0, 16)][0]

    col0 = pl.multiple_of(w * W, 128)

    def publish_all():
        off128 = pl.multiple_of(w * 128, 8)
        pltpu.sync_copy(sv_v, candv_h.at[pl.ds(off128, 128)])
        pltpu.sync_copy(si_v, candi_h.at[pl.ds(off128, 128)])

    @pl.when((mode == 0) & (w < NW - 1))
    def _scan_full():
        cl0 = pltpu.async_copy(
            log_h.at[pl.ds(0, B), pl.ds(col0, CH)],
            log_v.at[pl.ds(0, B), pl.ds(0, CH)], sem_l)
        cp0 = pltpu.async_copy(
            pen_h.at[pl.ds(0, B), pl.ds(col0, CH)],
            pen_v.at[pl.ds(0, B), pl.ds(0, CH)], sem_p)
        cl1 = pltpu.async_copy(
            log_h.at[pl.ds(0, B), pl.ds(col0 + CH, W - CH)],
            log_v.at[pl.ds(0, B), pl.ds(CH, W - CH)], sem_l2)
        cp1 = pltpu.async_copy(
            pen_h.at[pl.ds(0, B), pl.ds(col0 + CH, W - CH)],
            pen_v.at[pl.ds(0, B), pl.ds(CH, W - CH)], sem_p2)
        cl0.wait()
        cp0.wait()
        init = (neg_inf, jnp.zeros((16,), jnp.int32))
        carries = [
            _chunk_argmax(log_v, pen_v, r, col0, 0, CH, lane, init)
            for r in range(B)
        ]
        cl1.wait()
        cp1.wait()
        for r in range(B):
            bv, bi = _chunk_argmax(
                log_v, pen_v, r, col0, CH, W, lane, carries[r])
            sv_v[pl.ds(r * 16, 16)] = bv
            si_v[pl.ds(r * 16, 16)] = bi
        publish_all()

    @pl.when((mode == 0) & (w == NW - 1))
    def _scan_last():
        cl = pltpu.async_copy(
            log_h.at[pl.ds(0, B), pl.ds(col0, WLAST)],
            log_v.at[pl.ds(0, B), pl.ds(0, WLAST)], sem_l)
        cp = pltpu.async_copy(
            pen_h.at[pl.ds(0, B), pl.ds(col0, WLAST)],
            pen_v.at[pl.ds(0, B), pl.ds(0, WLAST)], sem_p)
        pltpu.sync_copy(log_h.at[pl.ds(0, B), pl.ds(EDGE0, EDGE)], lt_v)
        pltpu.sync_copy(pen_h.at[pl.ds(0, B), pl.ds(EDGE0, EDGE)], pt_v)
        cp.wait()
        cl.wait()
        for r in range(B):
            init = (neg_inf, jnp.zeros((16,), jnp.int32))
            bv, bi = _chunk_argmax(
                log_v, pen_v, r, col0, 0, WLAST, lane, init)
            for t in range(EDGE // 16):
                sc = lt_v[r, pl.ds(16 * t, 16)] * pt_v[r, pl.ds(16 * t, 16)]
                gi = EDGE0 + 16 * t + lane
                take = sc > bv
                bv = jnp.where(take, sc, bv)
                bi = jnp.where(take, gi, bi)
            sv_v[pl.ds(r * 16, 16)] = bv
            si_v[pl.ds(r * 16, 16)] = bi
        publish_all()

    @pl.when((mode == 1) & (s % 4 == 0))
    def _merge_patch():
        r0 = c * 4 + s // 4
        ca = pltpu.async_copy(candv_h, mv_v, sem_a)
        cb = pltpu.async_copy(candi_h, mi_v, sem_b)
        pltpu.sync_copy(pv_h, pv_v.at[pl.ds(0, 1)])
        ca.wait()
        cb.wait()
        pv_s = pv_v[pl.ds(0, 16)][0]

        wins = []
        for r in range(B):
            def merge(k, carry, r=r):
                bv, bi = carry
                off = k * 128 + r * 16
                cv = mv_v[pl.ds(off, 16)]
                ci = mi_v[pl.ds(off, 16)]
                take = (cv > bv) | ((cv == bv) & (ci < bi))
                return jnp.where(take, cv, bv), jnp.where(take, ci, bi)

            bv0 = mv_v[pl.ds(pl.multiple_of(r * 16, 16), 16)]
            bi0 = mi_v[pl.ds(pl.multiple_of(r * 16, 16), 16)]
            bestv, besti = lax.fori_loop(1, NW, merge, (bv0, bi0))

            stv_v[pl.ds(0, 16)] = bestv
            stv_v[pl.ds(16, 16)] = neg_inf
            sti_v[pl.ds(0, 16)] = besti
            sti_v[pl.ds(16, 16)] = jnp.full((16,), IMAX, dtype=jnp.int32)

            def red(i, carry):
                bv_s, bi_s = carry
                v = stv_v[pl.ds(i, 16)][0]
                ii = sti_v[pl.ds(i, 16)][0]
                better = (v > bv_s) | ((v == bv_s) & (ii < bi_s))
                return (jnp.where(better, v, bv_s),
                        jnp.where(better, ii, bi_s))

            _, win = lax.fori_loop(
                0, 16, red, (jnp.float32(-jnp.inf), jnp.int32(IMAX)))
            wins.append(win)

        tiles = [win // 128 for win in wins]

        win0 = wins[0]
        t0 = tiles[0]
        for r in range(1, B):
            win0 = jnp.where(r0 == r, wins[r], win0)
            t0 = jnp.where(r0 == r, tiles[r], t0)

        ist_v[...] = jnp.full((16,), win0, dtype=jnp.int32)
        pltpu.sync_copy(
            ist_v, idxw_h.at[pl.ds(pl.multiple_of(r0 * 16, 8), 16)])

        owner = jnp.bool_(True)
        for r in range(B):
            owner = owner & ((r >= r0) | (tiles[r] != t0))

        def patch_rows(buf):
            for r in range(B):
                @pl.when(tiles[r] == t0)
                def _(r=r):
                    woff = pl.multiple_of(
                        (wins[r] % 128) - (wins[r] % 16), 16)
                    tgt = wins[r] % 16
                    vec = buf[r, pl.ds(woff, 16)]
                    buf[r, pl.ds(woff, 16)] = jnp.where(
                        lane == tgt, vec * pv_s, vec)

        @pl.when(owner & (t0 < EDGE0 // 128))
        def _patch_main():
            ctile = pl.multiple_of(t0 * 128, 128)
            pltpu.sync_copy(out_h.at[pl.ds(0, B), pl.ds(ctile, 128)], pw_v)
            patch_rows(pw_v)
            pltpu.sync_copy(pw_v, out_h.at[pl.ds(0, B), pl.ds(ctile, 128)])

        @pl.when(owner & (t0 == EDGE0 // 128))
        def _patch_edge():
            pltpu.sync_copy(out_h.at[pl.ds(0, B), pl.ds(EDGE0, EDGE)], pe_v)
            patch_rows(pe_v)
            pltpu.sync_copy(pe_v, out_h.at[pl.ds(0, B), pl.ds(EDGE0, EDGE)])


@jax.jit
def _sc_greedy(logits, penalty, penval):
    mesh = plsc.VectorSubcoreMesh(core_axis_name="c", subcore_axis_name="s")
    ker = pl.kernel(
        _body,
        mesh=mesh,
        out_type=(),
        scratch_types=[
            pltpu.VMEM((B, W), jnp.float32),
            pltpu.VMEM((B, W), jnp.float32),
            pltpu.VMEM((B, EDGE), jnp.float32),
            pltpu.VMEM((B, EDGE), jnp.float32),
            pltpu.VMEM((128,), jnp.float32),
            pltpu.VMEM((128,), jnp.int32),
            pltpu.VMEM((NW * 128,), jnp.float32),
            pltpu.VMEM((NW * 128,), jnp.int32),
            pltpu.VMEM((32,), jnp.float32),
            pltpu.VMEM((32,), jnp.int32),
            pltpu.VMEM((16,), jnp.float32),
            pltpu.VMEM((B, 128), jnp.float32),
            pltpu.VMEM((B, EDGE), jnp.float32),
            pltpu.VMEM((16,), jnp.int32),
            pltpu.SemaphoreType.DMA,
            pltpu.SemaphoreType.DMA,
            pltpu.SemaphoreType.DMA,
            pltpu.SemaphoreType.DMA,
            pltpu.SemaphoreType.DMA,
            pltpu.SemaphoreType.DMA,
        ],
    )
    candv_ref = jax.new_ref(jnp.zeros((NW * 128,), jnp.float32))
    candi_ref = jax.new_ref(jnp.zeros((NW * 128,), jnp.int32))
    idxw_ref = jax.new_ref(jnp.zeros((B * 16,), jnp.int32))
    pen_ref = jax.new_ref(penalty)
    mode0 = jnp.zeros((1,), jnp.int32)
    mode1 = jnp.ones((1,), jnp.int32)
    ker(logits, penalty, penval, mode0,
        candv_ref, candi_ref, pen_ref, idxw_ref)
    ker(logits, penalty, penval, mode1,
        candv_ref, candi_ref, pen_ref, idxw_ref)
    return idxw_ref[...], pen_ref[...]


def kernel(logits, repeat_penality, penality_value, batch_size):
    del batch_size
    idx_wide, pen_out = _sc_greedy(logits, repeat_penality, penality_value)
    return idx_wide.reshape(B, 16)[:, :1], pen_out


# speedup vs baseline: 1.1638x; 1.1638x over previous
import jax
import jax.numpy as jnp
from jax import lax
from jax.experimental import pallas as pl
from jax.experimental.pallas import tpu as pltpu
from jax.experimental.pallas import tpu_sc as plsc

B = 8
V = 100000
W = 3200
WLAST = 768
EDGE0 = 31 * W + WLAST
EDGE = V - EDGE0
NW = 32
UNROLL = 8
IMAX = 2**31 - 1


def _chunk_argmax(log_v, pen_v, r, col0, lo, hi, lane, carry):
    def step(j, c):
        bv, bi = c
        off = pl.multiple_of(j * 16, 16)
        sc = log_v[r, pl.ds(off, 16)] * pen_v[r, pl.ds(off, 16)]
        gi = col0 + off + lane
        take = sc > bv
        return jnp.where(take, sc, bv), jnp.where(take, gi, bi)

    return lax.fori_loop(lo // 16, hi // 16, step, carry, unroll=UNROLL)


def _phase1(log_h, pen_h, candv_h, candi_h, out_h,
            log_v, pen_v, lt_v, pt_v, sv_v, si_v,
            sem_l, sem_p, sem_l2, sem_p2, sem_o):
    c = lax.axis_index("c")
    s = lax.axis_index("s")
    w = c * 16 + s
    col0 = pl.multiple_of(w * W, 128)
    lane = jnp.arange(16, dtype=jnp.int32)
    neg_inf = jnp.full((16,), -jnp.inf, dtype=jnp.float32)
    CH = 1536

    def publish_all():
        off128 = pl.multiple_of(w * 128, 8)
        pltpu.sync_copy(sv_v, candv_h.at[pl.ds(off128, 128)])
        pltpu.sync_copy(si_v, candi_h.at[pl.ds(off128, 128)])

    @pl.when(w < NW - 1)
    def _full():
        cl0 = pltpu.async_copy(
            log_h.at[pl.ds(0, B), pl.ds(col0, CH)],
            log_v.at[pl.ds(0, B), pl.ds(0, CH)], sem_l)
        cp0 = pltpu.async_copy(
            pen_h.at[pl.ds(0, B), pl.ds(col0, CH)],
            pen_v.at[pl.ds(0, B), pl.ds(0, CH)], sem_p)
        cl1 = pltpu.async_copy(
            log_h.at[pl.ds(0, B), pl.ds(col0 + CH, W - CH)],
            log_v.at[pl.ds(0, B), pl.ds(CH, W - CH)], sem_l2)
        cp1 = pltpu.async_copy(
            pen_h.at[pl.ds(0, B), pl.ds(col0 + CH, W - CH)],
            pen_v.at[pl.ds(0, B), pl.ds(CH, W - CH)], sem_p2)
        cl0.wait()
        cp0.wait()
        init = (neg_inf, jnp.zeros((16,), jnp.int32))
        carries = [
            _chunk_argmax(log_v, pen_v, r, col0, 0, CH, lane, init)
            for r in range(B)
        ]
        cl1.wait()
        cp1.wait()
        co = pltpu.async_copy(
            pen_v, out_h.at[pl.ds(0, B), pl.ds(col0, W)], sem_o)
        for r in range(B):
            bv, bi = _chunk_argmax(
                log_v, pen_v, r, col0, CH, W, lane, carries[r])
            sv_v[pl.ds(r * 16, 16)] = bv
            si_v[pl.ds(r * 16, 16)] = bi
        publish_all()
        co.wait()

    @pl.when(w == NW - 1)
    def _last():
        cl = pltpu.async_copy(
            log_h.at[pl.ds(0, B), pl.ds(col0, WLAST)],
            log_v.at[pl.ds(0, B), pl.ds(0, WLAST)], sem_l)
        cp = pltpu.async_copy(
            pen_h.at[pl.ds(0, B), pl.ds(col0, WLAST)],
            pen_v.at[pl.ds(0, B), pl.ds(0, WLAST)], sem_p)
        pltpu.sync_copy(log_h.at[pl.ds(0, B), pl.ds(EDGE0, EDGE)], lt_v)
        pltpu.sync_copy(pen_h.at[pl.ds(0, B), pl.ds(EDGE0, EDGE)], pt_v)
        cp.wait()
        co = pltpu.async_copy(
            pen_v.at[pl.ds(0, B), pl.ds(0, WLAST)],
            out_h.at[pl.ds(0, B), pl.ds(col0, WLAST)], sem_o)
        pltpu.sync_copy(pt_v, out_h.at[pl.ds(0, B), pl.ds(EDGE0, EDGE)])
        cl.wait()
        for r in range(B):
            init = (neg_inf, jnp.zeros((16,), jnp.int32))
            bv, bi = _chunk_argmax(
                log_v, pen_v, r, col0, 0, WLAST, lane, init)
            for t in range(EDGE // 16):
                sc = lt_v[r, pl.ds(16 * t, 16)] * pt_v[r, pl.ds(16 * t, 16)]
                gi = EDGE0 + 16 * t + lane
                take = sc > bv
                bv = jnp.where(take, sc, bv)
                bi = jnp.where(take, gi, bi)
            sv_v[pl.ds(r * 16, 16)] = bv
            si_v[pl.ds(r * 16, 16)] = bi
        publish_all()
        co.wait()


def _phase2(candv_h, candi_h, pv_h, pen_h, idxw_h,
            mv_v, mi_v, stv_v, sti_v, pv_v, pw_v, pe_v, ist_v,
            sem_a, sem_b, sem_c):
    c = lax.axis_index("c")
    s = lax.axis_index("s")
    lane = jnp.arange(16, dtype=jnp.int32)
    neg_inf = jnp.full((16,), -jnp.inf, dtype=jnp.float32)

    @pl.when(s % 4 == 0)
    def _active():
        r0 = c * 4 + s // 4
        ca = pltpu.async_copy(candv_h, mv_v, sem_a)
        cb = pltpu.async_copy(candi_h, mi_v, sem_b)
        pltpu.sync_copy(pv_h, pv_v.at[pl.ds(0, 1)])
        ca.wait()
        cb.wait()
        pv_s = pv_v[pl.ds(0, 16)][0]

        wins = []
        for r in range(B):
            def merge(k, carry, r=r):
                bv, bi = carry
                off = k * 128 + r * 16
                cv = mv_v[pl.ds(off, 16)]
                ci = mi_v[pl.ds(off, 16)]
                take = (cv > bv) | ((cv == bv) & (ci < bi))
                return jnp.where(take, cv, bv), jnp.where(take, ci, bi)

            bv0 = mv_v[pl.ds(pl.multiple_of(r * 16, 16), 16)]
            bi0 = mi_v[pl.ds(pl.multiple_of(r * 16, 16), 16)]
            bestv, besti = lax.fori_loop(1, NW, merge, (bv0, bi0))

            stv_v[pl.ds(0, 16)] = bestv
            stv_v[pl.ds(16, 16)] = neg_inf
            sti_v[pl.ds(0, 16)] = besti
            sti_v[pl.ds(16, 16)] = jnp.full((16,), IMAX, dtype=jnp.int32)

            def red(i, carry):
                bv_s, bi_s = carry
                v = stv_v[pl.ds(i, 16)][0]
                ii = sti_v[pl.ds(i, 16)][0]
                better = (v > bv_s) | ((v == bv_s) & (ii < bi_s))
                return (jnp.where(better, v, bv_s),
                        jnp.where(better, ii, bi_s))

            _, win = lax.fori_loop(
                0, 16, red, (jnp.float32(-jnp.inf), jnp.int32(IMAX)))
            wins.append(win)

        tiles = [win // 128 for win in wins]

        win0 = wins[0]
        t0 = tiles[0]
        for r in range(1, B):
            win0 = jnp.where(r0 == r, wins[r], win0)
            t0 = jnp.where(r0 == r, tiles[r], t0)

        ist_v[...] = jnp.full((16,), win0, dtype=jnp.int32)
        pltpu.sync_copy(
            ist_v, idxw_h.at[pl.ds(pl.multiple_of(r0 * 16, 8), 16)])

        owner = jnp.bool_(True)
        for r in range(B):
            owner = owner & ((r >= r0) | (tiles[r] != t0))

        def patch_rows(buf, width):
            for r in range(B):
                @pl.when(tiles[r] == t0)
                def _(r=r):
                    woff = pl.multiple_of(
                        (wins[r] % 128) - (wins[r] % 16), 16)
                    tgt = wins[r] % 16
                    vec = buf[r, pl.ds(woff, 16)]
                    buf[r, pl.ds(woff, 16)] = jnp.where(
                        lane == tgt, vec * pv_s, vec)

        @pl.when(owner & (t0 < EDGE0 // 128))
        def _patch_main():
            ctile = pl.multiple_of(t0 * 128, 128)
            pltpu.sync_copy(pen_h.at[pl.ds(0, B), pl.ds(ctile, 128)], pw_v)
            patch_rows(pw_v, 128)
            pltpu.sync_copy(pw_v, pen_h.at[pl.ds(0, B), pl.ds(ctile, 128)])

        @pl.when(owner & (t0 == EDGE0 // 128))
        def _patch_edge():
            pltpu.sync_copy(pen_h.at[pl.ds(0, B), pl.ds(EDGE0, EDGE)], pe_v)
            patch_rows(pe_v, EDGE)
            pltpu.sync_copy(pe_v, pen_h.at[pl.ds(0, B), pl.ds(EDGE0, EDGE)])


@jax.jit
def _sc_greedy(logits, penalty, penval):
    mesh = plsc.VectorSubcoreMesh(core_axis_name="c", subcore_axis_name="s")
    ker1 = pl.kernel(
        _phase1,
        mesh=mesh,
        out_type=[
            jax.ShapeDtypeStruct((NW * 128,), jnp.float32),
            jax.ShapeDtypeStruct((NW * 128,), jnp.int32),
            jax.ShapeDtypeStruct((B, V), jnp.float32),
        ],
        scratch_types=[
            pltpu.VMEM((B, W), jnp.float32),
            pltpu.VMEM((B, W), jnp.float32),
            pltpu.VMEM((B, EDGE), jnp.float32),
            pltpu.VMEM((B, EDGE), jnp.float32),
            pltpu.VMEM((128,), jnp.float32),
            pltpu.VMEM((128,), jnp.int32),
            pltpu.SemaphoreType.DMA,
            pltpu.SemaphoreType.DMA,
            pltpu.SemaphoreType.DMA,
            pltpu.SemaphoreType.DMA,
            pltpu.SemaphoreType.DMA,
        ],
    )
    cand_v, cand_i, pen_out = ker1(logits, penalty)

    pen_ref = jax.new_ref(pen_out)
    ker2 = pl.kernel(
        _phase2,
        mesh=mesh,
        out_type=jax.ShapeDtypeStruct((B * 16,), jnp.int32),
        scratch_types=[
            pltpu.VMEM((NW * 128,), jnp.float32),
            pltpu.VMEM((NW * 128,), jnp.int32),
            pltpu.VMEM((32,), jnp.float32),
            pltpu.VMEM((32,), jnp.int32),
            pltpu.VMEM((16,), jnp.float32),
            pltpu.VMEM((B, 128), jnp.float32),
            pltpu.VMEM((B, EDGE), jnp.float32),
            pltpu.VMEM((16,), jnp.int32),
            pltpu.SemaphoreType.DMA,
            pltpu.SemaphoreType.DMA,
            pltpu.SemaphoreType.DMA,
        ],
    )
    idx_wide = ker2(cand_v, cand_i, penval, pen_ref)
    return idx_wide, pen_ref[...]


def kernel(logits, repeat_penality, penality_value, batch_size):
    del batch_size
    idx_wide, pen_out = _sc_greedy(logits, repeat_penality, penality_value)
    return idx_wide.reshape(B, 16)[:, :1], pen_out
